# Initial kernel scaffold; baseline (speedup 1.0000x reference)
import jax
import jax.numpy as jnp
from jax.experimental import pallas as pl


def kernel(nodes_emb, edge_weight, b, edge_index):
    raise NotImplementedError("write your pallas kernel here")


# speedup vs baseline: 2.8881x; 2.8881x over previous
import functools

import jax
import jax.numpy as jnp
from jax import lax
from jax.experimental import pallas as pl
from jax.experimental.pallas import tpu as pltpu
from jax.experimental.pallas import tpu_sc as plsc

N_NODES = 10000
N_PAD = 10240
EMB = 128
N_EDGES = 320000
NC = 2
NS = 16
HALF = N_PAD // NC
TRASH = 128
ACC_ROWS = HALF + TRASH
EDGES_PER_TILE = N_EDGES // NS
CHUNK = 128
N_CHUNKS = -(-EDGES_PER_TILE // CHUNK)
EDGES_PER_TILE_PAD = N_CHUNKS * CHUNK


def _spmm_sc(x, src_t, dst_t, w_t):
  mesh = plsc.VectorSubcoreMesh(core_axis_name="c", subcore_axis_name="s")

  @functools.partial(
      pl.kernel,
      mesh=mesh,
      out_type=jax.ShapeDtypeStruct((N_PAD, EMB), jnp.float32),
      scratch_types=[
          pltpu.VMEM((N_CHUNKS, CHUNK), jnp.int32),
          pltpu.VMEM((N_CHUNKS, CHUNK), jnp.int32),
          pltpu.VMEM((N_CHUNKS, CHUNK), jnp.float32),
          pltpu.VMEM((CHUNK, EMB), jnp.float32),
          pltpu.VMEM_SHARED((ACC_ROWS, EMB), jnp.float32),
          pltpu.SemaphoreType.DMA,
      ],
  )
  def spmm(x_hbm, src_hbm, dst_hbm, w_hbm, out_hbm,
           src_v, dst_v, w_v, rows_v, acc_sh, sem):
    c = lax.axis_index("c")
    s = lax.axis_index("s")

    z16 = jnp.zeros((16,), jnp.float32)

    def zfill(i, _):
      r = i // (EMB // 16)
      j = i % (EMB // 16)
      rows_v[r, pl.ds(j * 16, 16)] = z16
      return 0

    lax.fori_loop(0, CHUNK * (EMB // 16), zfill, 0)
    rows_per_tile = ACC_ROWS // NS
    base = s * rows_per_tile
    pltpu.sync_copy(rows_v, acc_sh.at[pl.ds(base, CHUNK)])
    pltpu.sync_copy(rows_v, acc_sh.at[pl.ds(base + CHUNK, CHUNK)])
    pltpu.sync_copy(rows_v.at[pl.ds(0, rows_per_tile - 2 * CHUNK)],
                    acc_sh.at[pl.ds(base + 2 * CHUNK, rows_per_tile - 2 * CHUNK)])

    pltpu.sync_copy(src_hbm.at[s], src_v)
    pltpu.sync_copy(dst_hbm.at[s], dst_v)
    pltpu.sync_copy(w_hbm.at[s], w_v)

    lo = c * HALF

    def remap_body(i, _):
      k = i // (CHUNK // 16)
      g = i % (CHUNK // 16)
      sl = pl.ds(g * 16, 16)
      d = dst_v[k, sl] - lo
      valid = (d >= 0) & (d < HALF)
      dst_v[k, sl] = jnp.where(valid, d, HALF)
      return 0

    lax.fori_loop(0, N_CHUNKS * (CHUNK // 16), remap_body, 0)
    plsc.subcore_barrier()

    def chunk_body(k, _):
      pltpu.async_copy(x_hbm.at[src_v.at[k]], rows_v, sem).wait()

      def group_body(g, _):
        wg = w_v[k, pl.ds(g * 16, 16)]
        for e in range(16):
          we = wg[e]
          row = g * 16 + e
          for j in range(EMB // 16):
            sl = pl.ds(j * 16, 16)
            rows_v[row, sl] = rows_v[row, sl] * we
        return 0

      lax.fori_loop(0, CHUNK // 16, group_body, 0)

      pltpu.sync_copy(rows_v, acc_sh.at[dst_v.at[k]], add=True)
      return 0

    lax.fori_loop(0, N_CHUNKS, chunk_body, 0)
    plsc.subcore_barrier()

    out_rows = HALF // NS
    pltpu.sync_copy(acc_sh.at[pl.ds(s * out_rows, out_rows)],
                    out_hbm.at[pl.ds(c * HALF + s * out_rows, out_rows)])

  return spmm(x, src_t, dst_t, w_t)


_ROWS_BLK = 1024


def _finalize(bvec, x0, x1, x2):
  def body(b_ref, x0_ref, x1_ref, x2_ref, o_ref):
    def n(v):
      ss = jnp.sum(v * v, axis=-1, keepdims=True)
      nrm = jnp.sqrt(ss)
      return v / jnp.maximum(nrm, 1e-12)

    acc = (b_ref[0] * n(x0_ref[...]) + b_ref[1] * n(x1_ref[...])
           + b_ref[2] * n(x2_ref[...]))
    o_ref[...] = 2.0 * acc

  blk = lambda: pl.BlockSpec((_ROWS_BLK, EMB), lambda i: (i, 0))
  return pl.pallas_call(
      body,
      grid=(N_PAD // _ROWS_BLK,),
      in_specs=[pl.BlockSpec(memory_space=pltpu.SMEM), blk(), blk(), blk()],
      out_specs=blk(),
      out_shape=jax.ShapeDtypeStruct((N_PAD, EMB), jnp.float32),
  )(bvec, x0, x1, x2)


def kernel(nodes_emb, edge_weight, b, edge_index):
  pad = NS * EDGES_PER_TILE_PAD - N_EDGES

  def tile_edges(a, fill):
    a = a.reshape(NS, EDGES_PER_TILE)
    a = jnp.pad(a, ((0, 0), (0, EDGES_PER_TILE_PAD - EDGES_PER_TILE)),
                constant_values=fill)
    return a.reshape(NS, N_CHUNKS, CHUNK)

  src_t = tile_edges(edge_index[0], 0)
  dst_t = tile_edges(edge_index[1], N_PAD)
  w_t = tile_edges(edge_weight, 0.0)
  bvec = b.reshape(3)

  x0 = jnp.pad(nodes_emb, ((0, N_PAD - N_NODES), (0, 0)))
  x1 = _spmm_sc(x0, src_t, dst_t, w_t)
  x2 = _spmm_sc(x1, src_t, dst_t, w_t)
  core = _finalize(bvec, x0, x1, x2)
  zeros = jnp.zeros((1, EMB), jnp.float32)
  return jnp.concatenate([zeros, core[:N_NODES]], axis=0)
